# baseline (speedup 1.0000x reference)
import functools

import jax
import jax.numpy as jnp
from jax import lax
from jax.experimental import pallas as pl
from jax.experimental.pallas import tpu as pltpu
from jax.experimental.pallas import tpu_sc as plsc

N_NODES_C = 10000
N_EDGES_C = 320000
D_C = 128

NC = 2
NS = 16
NW = NC * NS

CH = 128
NCHUNK = 80
EDGES_PER_TILE = CH * NCHUNK
E_PAD = EDGES_PER_TILE * NW
PACK = 16384
ACC_ROWS = 10112
ROWS_PER_TILE = ACC_ROWS // NS


def _sc_body(idx_hbm, hn_hbm, zeros_hbm, out_hbm,
             pk_v, sbuf, dbuf, rows_v, acc, sg0, sg1, ss0, ss1):
    c = lax.axis_index("c")
    s = lax.axis_index("s")
    wid = c * NS + s

    row0 = s * ROWS_PER_TILE
    pltpu.sync_copy(zeros_hbm.at[pl.ds(row0, ROWS_PER_TILE)],
                    acc.at[pl.ds(row0, ROWS_PER_TILE)])
    pltpu.sync_copy(idx_hbm.at[wid], pk_v)
    plsc.subcore_barrier()

    def unpack(j, b):
        for i in range(CH // 16):
            p = pk_v[j, pl.ds(16 * i, 16)]
            sbuf[b, pl.ds(16 * i, 16)] = lax.shift_right_logical(p, 14)
            dbuf[b, pl.ds(16 * i, 16)] = lax.bitwise_and(p, PACK - 1)

    def g_start(b, sg):
        pltpu.async_copy(hn_hbm.at[sbuf.at[b]], rows_v.at[b], sg)

    def g_wait(b, sg):
        pltpu.make_async_copy(hn_hbm.at[sbuf.at[b]], rows_v.at[b],
                              sg).wait()

    def s_start(b, ss):
        pltpu.async_copy(rows_v.at[b], acc.at[dbuf.at[b]], ss, add=True)

    def s_wait(b, ss):
        pltpu.make_async_copy(rows_v.at[b], acc.at[dbuf.at[b]],
                              ss).wait()

    unpack(0, 0)
    unpack(1, 1)
    g_start(0, sg0)
    g_start(1, sg1)

    def pair(k, carry):
        j0 = 2 * k
        g_wait(0, sg0)
        s_start(0, ss0)
        g_wait(1, sg1)
        s_start(1, ss1)
        s_wait(0, ss0)
        unpack(j0 + 2, 0)
        g_start(0, sg0)
        s_wait(1, ss1)
        unpack(j0 + 3, 1)
        g_start(1, sg1)
        return carry

    lax.fori_loop(0, NCHUNK // 2, pair, 0, unroll=False)
    g_wait(0, sg0)
    g_wait(1, sg1)

    plsc.subcore_barrier()
    pltpu.sync_copy(acc.at[pl.ds(row0, ROWS_PER_TILE)],
                    out_hbm.at[c, pl.ds(row0, ROWS_PER_TILE)])


@functools.cache
def _sc_aggregate():
    return functools.partial(
        pl.kernel,
        out_type=jax.ShapeDtypeStruct((NC, ACC_ROWS, D_C), jnp.float32),
        mesh=plsc.VectorSubcoreMesh(core_axis_name="c", subcore_axis_name="s",
                                    num_cores=NC, num_subcores=NS),
        scratch_types=[
            pltpu.VMEM((NCHUNK + 2, CH), jnp.int32),
            pltpu.VMEM((2, CH), jnp.int32),
            pltpu.VMEM((2, CH), jnp.int32),
            pltpu.VMEM((2, CH, D_C), jnp.float32),
            pltpu.VMEM_SHARED((ACC_ROWS, D_C), jnp.float32),
            pltpu.SemaphoreType.DMA,
            pltpu.SemaphoreType.DMA,
            pltpu.SemaphoreType.DMA,
            pltpu.SemaphoreType.DMA,
        ],
    )(_sc_body)


def _mlp_body(hn_ref, p_ref, w1a_ref, w1b_ref, b1_ref, w2_ref, b2_ref,
              w3_ref, b3_ref, out_ref):
    aggr = p_ref[0] + p_ref[1]
    hi = lax.Precision.HIGHEST
    h = jnp.dot(hn_ref[...], w1a_ref[...], precision=hi,
                preferred_element_type=jnp.float32)
    h += jnp.dot(aggr, w1b_ref[...], precision=hi,
                 preferred_element_type=jnp.float32)
    h = jnp.maximum(h + b1_ref[...], 0.0)
    h = jnp.maximum(
        jnp.dot(h, w2_ref[...], precision=hi,
                preferred_element_type=jnp.float32) + b2_ref[...], 0.0)
    out_ref[...] = jnp.dot(h, w3_ref[...], precision=hi,
                           preferred_element_type=jnp.float32) + b3_ref[...]


def _mlp(hn, partials, w1a, w1b, b1, w2, b2, w3, b3):
    blk = 2000
    grid = (N_NODES_C // blk,)
    wspec = pl.BlockSpec((D_C, D_C), lambda i: (0, 0))
    bspec = pl.BlockSpec((1, D_C), lambda i: (0, 0))
    return pl.pallas_call(
        _mlp_body,
        grid=grid,
        in_specs=[
            pl.BlockSpec((blk, D_C), lambda i: (i, 0)),
            pl.BlockSpec((NC, blk, D_C), lambda i: (0, i, 0)),
            wspec, wspec, bspec, wspec, bspec, wspec, bspec,
        ],
        out_specs=pl.BlockSpec((blk, D_C), lambda i: (i, 0)),
        out_shape=jax.ShapeDtypeStruct((N_NODES_C, D_C), jnp.float32),
    )(hn, partials, w1a, w1b, b1, w2, b2, w3, b3)


def kernel(hn, edge_index, he, W1, b1, W2, b2, W3, b3):
    del he
    src = edge_index[0]
    dst = edge_index[1]
    pad = E_PAD - N_EDGES_C
    src = jnp.concatenate([src, jnp.zeros((pad,), jnp.int32)])
    dst = jnp.concatenate([dst, jnp.full((pad,), N_NODES_C, jnp.int32)])
    packed = (src * PACK + dst).reshape(NW, NCHUNK, CH)
    dummy = jnp.full((NW, 2, CH), N_NODES_C, jnp.int32)
    idx = jnp.concatenate([packed, dummy], axis=1)
    zeros = jnp.zeros((ACC_ROWS, D_C), jnp.float32)

    partials = _sc_aggregate()(idx, hn, zeros)

    w1a = W1[:, :D_C].T
    w1b = W1[:, D_C:].T
    return _mlp(hn, partials, w1a, w1b, b1.reshape(1, -1),
                W2.T, b2.reshape(1, -1), W3.T, b3.reshape(1, -1))


# speedup vs baseline: 1.0165x; 1.0165x over previous
import functools

import jax
import jax.numpy as jnp
from jax import lax
from jax.experimental import pallas as pl
from jax.experimental.pallas import tpu as pltpu
from jax.experimental.pallas import tpu_sc as plsc

N_NODES_C = 10000
N_EDGES_C = 320000
D_C = 128

NC = 2
NS = 16
NW = NC * NS

CH = 128
NCHUNK = 80
EDGES_PER_TILE = CH * NCHUNK
E_PAD = EDGES_PER_TILE * NW
PACK = 16384
ACC_ROWS = 10112
ROWS_PER_TILE = ACC_ROWS // NS


def _sc_body(idx_hbm, hn_hbm, zeros_hbm, out_hbm,
             pk_v, sbuf, dbuf, rows_v, acc, sg0, sg1, ss0, ss1):
    c = lax.axis_index("c")
    s = lax.axis_index("s")
    wid = c * NS + s

    row0 = s * ROWS_PER_TILE
    pltpu.sync_copy(zeros_hbm.at[pl.ds(row0, ROWS_PER_TILE)],
                    acc.at[pl.ds(row0, ROWS_PER_TILE)])
    pltpu.sync_copy(idx_hbm.at[wid], pk_v)
    plsc.subcore_barrier()

    def unpack(j, b):
        for i in range(CH // 16):
            p = pk_v[j, pl.ds(16 * i, 16)]
            sbuf[b, pl.ds(16 * i, 16)] = lax.shift_right_logical(p, 14)
            dbuf[b, pl.ds(16 * i, 16)] = lax.bitwise_and(p, PACK - 1)

    def g_start(b, sg):
        pltpu.async_copy(hn_hbm.at[sbuf.at[b]], rows_v.at[b], sg)

    def g_wait(b, sg):
        pltpu.make_async_copy(hn_hbm.at[sbuf.at[b]], rows_v.at[b],
                              sg).wait()

    def s_start(b, ss):
        pltpu.async_copy(rows_v.at[b], acc.at[dbuf.at[b]], ss, add=True)

    def s_wait(b, ss):
        pltpu.make_async_copy(rows_v.at[b], acc.at[dbuf.at[b]],
                              ss).wait()

    unpack(0, 0)
    unpack(1, 1)
    g_start(0, sg0)
    g_start(1, sg1)

    def pair(k, carry):
        j0 = 2 * k
        g_wait(0, sg0)
        pltpu.sync_copy(rows_v.at[0], acc.at[dbuf.at[0]], add=True)
        unpack(j0 + 2, 0)
        g_start(0, sg0)
        g_wait(1, sg1)
        pltpu.sync_copy(rows_v.at[1], acc.at[dbuf.at[1]], add=True)
        unpack(j0 + 3, 1)
        g_start(1, sg1)
        return carry

    lax.fori_loop(0, NCHUNK // 2, pair, 0, unroll=False)
    g_wait(0, sg0)
    g_wait(1, sg1)

    plsc.subcore_barrier()
    pltpu.sync_copy(acc.at[pl.ds(row0, ROWS_PER_TILE)],
                    out_hbm.at[c, pl.ds(row0, ROWS_PER_TILE)])


@functools.cache
def _sc_aggregate():
    return functools.partial(
        pl.kernel,
        out_type=jax.ShapeDtypeStruct((NC, ACC_ROWS, D_C), jnp.float32),
        mesh=plsc.VectorSubcoreMesh(core_axis_name="c", subcore_axis_name="s",
                                    num_cores=NC, num_subcores=NS),
        scratch_types=[
            pltpu.VMEM((NCHUNK + 2, CH), jnp.int32),
            pltpu.VMEM((2, CH), jnp.int32),
            pltpu.VMEM((2, CH), jnp.int32),
            pltpu.VMEM((2, CH, D_C), jnp.float32),
            pltpu.VMEM_SHARED((ACC_ROWS, D_C), jnp.float32),
            pltpu.SemaphoreType.DMA,
            pltpu.SemaphoreType.DMA,
            pltpu.SemaphoreType.DMA,
            pltpu.SemaphoreType.DMA,
        ],
    )(_sc_body)


def _mlp_body(hn_ref, p_ref, w1a_ref, w1b_ref, b1_ref, w2_ref, b2_ref,
              w3_ref, b3_ref, out_ref):
    aggr = p_ref[0] + p_ref[1]
    hi = lax.Precision.HIGHEST
    h = jnp.dot(hn_ref[...], w1a_ref[...], precision=hi,
                preferred_element_type=jnp.float32)
    h += jnp.dot(aggr, w1b_ref[...], precision=hi,
                 preferred_element_type=jnp.float32)
    h = jnp.maximum(h + b1_ref[...], 0.0)
    h = jnp.maximum(
        jnp.dot(h, w2_ref[...], precision=hi,
                preferred_element_type=jnp.float32) + b2_ref[...], 0.0)
    out_ref[...] = jnp.dot(h, w3_ref[...], precision=hi,
                           preferred_element_type=jnp.float32) + b3_ref[...]


def _mlp(hn, partials, w1a, w1b, b1, w2, b2, w3, b3):
    blk = 2000
    grid = (N_NODES_C // blk,)
    wspec = pl.BlockSpec((D_C, D_C), lambda i: (0, 0))
    bspec = pl.BlockSpec((1, D_C), lambda i: (0, 0))
    return pl.pallas_call(
        _mlp_body,
        grid=grid,
        in_specs=[
            pl.BlockSpec((blk, D_C), lambda i: (i, 0)),
            pl.BlockSpec((NC, blk, D_C), lambda i: (0, i, 0)),
            wspec, wspec, bspec, wspec, bspec, wspec, bspec,
        ],
        out_specs=pl.BlockSpec((blk, D_C), lambda i: (i, 0)),
        out_shape=jax.ShapeDtypeStruct((N_NODES_C, D_C), jnp.float32),
    )(hn, partials, w1a, w1b, b1, w2, b2, w3, b3)


def kernel(hn, edge_index, he, W1, b1, W2, b2, W3, b3):
    del he
    src = edge_index[0]
    dst = edge_index[1]
    pad = E_PAD - N_EDGES_C
    src = jnp.concatenate([src, jnp.zeros((pad,), jnp.int32)])
    dst = jnp.concatenate([dst, jnp.full((pad,), N_NODES_C, jnp.int32)])
    packed = (src * PACK + dst).reshape(NW, NCHUNK, CH)
    dummy = jnp.full((NW, 2, CH), N_NODES_C, jnp.int32)
    idx = jnp.concatenate([packed, dummy], axis=1)
    zeros = jnp.zeros((ACC_ROWS, D_C), jnp.float32)

    partials = _sc_aggregate()(idx, hn, zeros)

    w1a = W1[:, :D_C].T
    w1b = W1[:, D_C:].T
    return _mlp(hn, partials, w1a, w1b, b1.reshape(1, -1),
                W2.T, b2.reshape(1, -1), W3.T, b3.reshape(1, -1))


# speedup vs baseline: 1.4459x; 1.4225x over previous
import functools

import jax
import jax.numpy as jnp
from jax import lax
from jax.experimental import pallas as pl
from jax.experimental.pallas import tpu as pltpu
from jax.experimental.pallas import tpu_sc as plsc

N_NODES_C = 10000
N_EDGES_C = 320000
D_C = 128

NC = 2
NS = 16
NW = NC * NS

CH = 128
NCHUNK = 80
EDGES_PER_TILE = CH * NCHUNK
E_PAD = EDGES_PER_TILE * NW
PACK = 16384
ACC_ROWS = 10112
ROWS_PER_TILE = ACC_ROWS // NS


def _sc_body(idx_hbm, hn_hbm, zeros_hbm, out_hbm,
             pk_v, sbuf, dbuf, rows_v, acc, sg0, sg1, ss0, ss1):
    c = lax.axis_index("c")
    s = lax.axis_index("s")
    wid = c * NS + s

    row0 = s * ROWS_PER_TILE
    pltpu.sync_copy(zeros_hbm.at[pl.ds(row0, ROWS_PER_TILE)],
                    acc.at[pl.ds(row0, ROWS_PER_TILE)])
    pltpu.sync_copy(idx_hbm.at[wid], pk_v)
    plsc.subcore_barrier()

    def unpack(j, b):
        for i in range(CH // 16):
            p = pk_v[j, pl.ds(16 * i, 16)]
            sbuf[b, pl.ds(16 * i, 16)] = lax.shift_right_logical(p, 14)
            dbuf[b, pl.ds(16 * i, 16)] = lax.bitwise_and(p, PACK - 1)

    def g_start(b, sg):
        pltpu.async_copy(hn_hbm.at[sbuf.at[b]], rows_v.at[b], sg)

    def g_wait(b, sg):
        pltpu.make_async_copy(hn_hbm.at[sbuf.at[b]], rows_v.at[b],
                              sg).wait()

    def s_start(b, ss):
        pltpu.async_copy(rows_v.at[b], acc.at[dbuf.at[b]], ss, add=True)

    def s_wait(b, ss):
        pltpu.make_async_copy(rows_v.at[b], acc.at[dbuf.at[b]],
                              ss).wait()

    def chunk(j, carry):
        unpack(j, 0)
        pltpu.async_copy(hn_hbm.at[sbuf.at[0]], rows_v.at[0], sg0).wait()
        pltpu.sync_copy(rows_v.at[0], acc.at[dbuf.at[0]], add=True)
        return carry

    lax.fori_loop(0, NCHUNK, chunk, 0, unroll=False)

    plsc.subcore_barrier()
    pltpu.sync_copy(acc.at[pl.ds(row0, ROWS_PER_TILE)],
                    out_hbm.at[c, pl.ds(row0, ROWS_PER_TILE)])


@functools.cache
def _sc_aggregate():
    return functools.partial(
        pl.kernel,
        out_type=jax.ShapeDtypeStruct((NC, ACC_ROWS, D_C), jnp.float32),
        mesh=plsc.VectorSubcoreMesh(core_axis_name="c", subcore_axis_name="s",
                                    num_cores=NC, num_subcores=NS),
        scratch_types=[
            pltpu.VMEM((NCHUNK + 2, CH), jnp.int32),
            pltpu.VMEM((2, CH), jnp.int32),
            pltpu.VMEM((2, CH), jnp.int32),
            pltpu.VMEM((2, CH, D_C), jnp.float32),
            pltpu.VMEM_SHARED((ACC_ROWS, D_C), jnp.float32),
            pltpu.SemaphoreType.DMA,
            pltpu.SemaphoreType.DMA,
            pltpu.SemaphoreType.DMA,
            pltpu.SemaphoreType.DMA,
        ],
    )(_sc_body)


def _mlp_body(hn_ref, p_ref, w1a_ref, w1b_ref, b1_ref, w2_ref, b2_ref,
              w3_ref, b3_ref, out_ref):
    aggr = p_ref[0] + p_ref[1]
    hi = lax.Precision.HIGHEST
    h = jnp.dot(hn_ref[...], w1a_ref[...], precision=hi,
                preferred_element_type=jnp.float32)
    h += jnp.dot(aggr, w1b_ref[...], precision=hi,
                 preferred_element_type=jnp.float32)
    h = jnp.maximum(h + b1_ref[...], 0.0)
    h = jnp.maximum(
        jnp.dot(h, w2_ref[...], precision=hi,
                preferred_element_type=jnp.float32) + b2_ref[...], 0.0)
    out_ref[...] = jnp.dot(h, w3_ref[...], precision=hi,
                           preferred_element_type=jnp.float32) + b3_ref[...]


def _mlp(hn, partials, w1a, w1b, b1, w2, b2, w3, b3):
    blk = 2000
    grid = (N_NODES_C // blk,)
    wspec = pl.BlockSpec((D_C, D_C), lambda i: (0, 0))
    bspec = pl.BlockSpec((1, D_C), lambda i: (0, 0))
    return pl.pallas_call(
        _mlp_body,
        grid=grid,
        in_specs=[
            pl.BlockSpec((blk, D_C), lambda i: (i, 0)),
            pl.BlockSpec((NC, blk, D_C), lambda i: (0, i, 0)),
            wspec, wspec, bspec, wspec, bspec, wspec, bspec,
        ],
        out_specs=pl.BlockSpec((blk, D_C), lambda i: (i, 0)),
        out_shape=jax.ShapeDtypeStruct((N_NODES_C, D_C), jnp.float32),
    )(hn, partials, w1a, w1b, b1, w2, b2, w3, b3)


def kernel(hn, edge_index, he, W1, b1, W2, b2, W3, b3):
    del he
    src = edge_index[0]
    dst = edge_index[1]
    pad = E_PAD - N_EDGES_C
    src = jnp.concatenate([src, jnp.zeros((pad,), jnp.int32)])
    dst = jnp.concatenate([dst, jnp.full((pad,), N_NODES_C, jnp.int32)])
    packed = (src * PACK + dst).reshape(NW, NCHUNK, CH)
    dummy = jnp.full((NW, 2, CH), N_NODES_C, jnp.int32)
    idx = jnp.concatenate([packed, dummy], axis=1)
    zeros = jnp.zeros((ACC_ROWS, D_C), jnp.float32)

    partials = _sc_aggregate()(idx, hn, zeros)

    w1a = W1[:, :D_C].T
    w1b = W1[:, D_C:].T
    return _mlp(hn, partials, w1a, w1b, b1.reshape(1, -1),
                W2.T, b2.reshape(1, -1), W3.T, b3.reshape(1, -1))


# speedup vs baseline: 1.5152x; 1.0479x over previous
import functools

import jax
import jax.numpy as jnp
from jax import lax
from jax.experimental import pallas as pl
from jax.experimental.pallas import tpu as pltpu
from jax.experimental.pallas import tpu_sc as plsc

N_NODES_C = 10000
N_EDGES_C = 320000
D_C = 128

NC = 2
NS = 16
NW = NC * NS

CH = 128
NCHUNK = 80
EDGES_PER_TILE = CH * NCHUNK
E_PAD = EDGES_PER_TILE * NW
PACK = 16384
ACC_ROWS = 10112
ROWS_PER_TILE = ACC_ROWS // NS


def _sc_body(src_hbm, dst_hbm, hn_hbm, zeros_hbm, out_hbm,
             src_v, dst_v, rows_v, acc, sg0):
    c = lax.axis_index("c")
    s = lax.axis_index("s")
    wid = c * NS + s

    row0 = s * ROWS_PER_TILE
    pltpu.sync_copy(zeros_hbm.at[pl.ds(row0, ROWS_PER_TILE)],
                    acc.at[pl.ds(row0, ROWS_PER_TILE)])
    pltpu.sync_copy(src_hbm.at[wid], src_v)
    pltpu.sync_copy(dst_hbm.at[wid], dst_v)
    plsc.subcore_barrier()

    def chunk(j, carry):
        pltpu.async_copy(hn_hbm.at[src_v.at[j]], rows_v, sg0).wait()
        pltpu.sync_copy(rows_v, acc.at[dst_v.at[j]], add=True)
        return carry

    lax.fori_loop(0, NCHUNK, chunk, 0, unroll=False)

    plsc.subcore_barrier()
    pltpu.sync_copy(acc.at[pl.ds(row0, ROWS_PER_TILE)],
                    out_hbm.at[c, pl.ds(row0, ROWS_PER_TILE)])


@functools.cache
def _sc_aggregate():
    return functools.partial(
        pl.kernel,
        out_type=jax.ShapeDtypeStruct((NC, ACC_ROWS, D_C), jnp.float32),
        mesh=plsc.VectorSubcoreMesh(core_axis_name="c", subcore_axis_name="s",
                                    num_cores=NC, num_subcores=NS),
        scratch_types=[
            pltpu.VMEM((NCHUNK, CH), jnp.int32),
            pltpu.VMEM((NCHUNK, CH), jnp.int32),
            pltpu.VMEM((CH, D_C), jnp.float32),
            pltpu.VMEM_SHARED((ACC_ROWS, D_C), jnp.float32),
            pltpu.SemaphoreType.DMA,
        ],
    )(_sc_body)


def _mlp_body(hn_ref, p_ref, w1a_ref, w1b_ref, b1_ref, w2_ref, b2_ref,
              w3_ref, b3_ref, out_ref):
    aggr = p_ref[0] + p_ref[1]
    hi = lax.Precision.DEFAULT
    h = jnp.dot(hn_ref[...], w1a_ref[...], precision=hi,
                preferred_element_type=jnp.float32)
    h += jnp.dot(aggr, w1b_ref[...], precision=hi,
                 preferred_element_type=jnp.float32)
    h = jnp.maximum(h + b1_ref[...], 0.0)
    h = jnp.maximum(
        jnp.dot(h, w2_ref[...], precision=hi,
                preferred_element_type=jnp.float32) + b2_ref[...], 0.0)
    out_ref[...] = jnp.dot(h, w3_ref[...], precision=hi,
                           preferred_element_type=jnp.float32) + b3_ref[...]


def _mlp(hn, partials, w1a, w1b, b1, w2, b2, w3, b3):
    blk = 2000
    grid = (N_NODES_C // blk,)
    wspec = pl.BlockSpec((D_C, D_C), lambda i: (0, 0))
    bspec = pl.BlockSpec((1, D_C), lambda i: (0, 0))
    return pl.pallas_call(
        _mlp_body,
        grid=grid,
        in_specs=[
            pl.BlockSpec((blk, D_C), lambda i: (i, 0)),
            pl.BlockSpec((NC, blk, D_C), lambda i: (0, i, 0)),
            wspec, wspec, bspec, wspec, bspec, wspec, bspec,
        ],
        out_specs=pl.BlockSpec((blk, D_C), lambda i: (i, 0)),
        out_shape=jax.ShapeDtypeStruct((N_NODES_C, D_C), jnp.float32),
    )(hn, partials, w1a, w1b, b1, w2, b2, w3, b3)


def kernel(hn, edge_index, he, W1, b1, W2, b2, W3, b3):
    del he
    src = edge_index[0]
    dst = edge_index[1]
    pad = E_PAD - N_EDGES_C
    src = jnp.concatenate([src, jnp.zeros((pad,), jnp.int32)])
    dst = jnp.concatenate([dst, jnp.full((pad,), N_NODES_C, jnp.int32)])
    src = src.reshape(NW, NCHUNK, CH)
    dst = dst.reshape(NW, NCHUNK, CH)
    zeros = jnp.zeros((ACC_ROWS, D_C), jnp.float32)

    partials = _sc_aggregate()(src, dst, hn, zeros)

    w1a = W1[:, :D_C].T
    w1b = W1[:, D_C:].T
    return _mlp(hn, partials, w1a, w1b, b1.reshape(1, -1),
                W2.T, b2.reshape(1, -1), W3.T, b3.reshape(1, -1))


# speedup vs baseline: 2.2826x; 1.5064x over previous
import functools

import jax
import jax.numpy as jnp
from jax import lax
from jax.experimental import pallas as pl
from jax.experimental.pallas import tpu as pltpu
from jax.experimental.pallas import tpu_sc as plsc

N_NODES_C = 10000
N_EDGES_C = 320000
D_C = 128

NC = 2
NS = 16
NW = NC * NS

CH = 128
NCHUNK = 79
EDGES_PER_TILE = CH * NCHUNK
E_PAD = EDGES_PER_TILE * NW
ACC_ROWS = 10240
ROWS_PER_TILE = ACC_ROWS // NS


def _sc_body(src_hbm, dst_hbm, hn_hbm, zeros_hbm, out_hbm,
             src_v, dst_v, rows_v, acc, sg0):
    c = lax.axis_index("c")
    s = lax.axis_index("s")
    wid = c * NS + s

    row0 = s * ROWS_PER_TILE
    pltpu.sync_copy(zeros_hbm.at[pl.ds(row0, ROWS_PER_TILE)],
                    acc.at[pl.ds(row0, ROWS_PER_TILE)])
    pltpu.sync_copy(src_hbm.at[wid], src_v)
    pltpu.sync_copy(dst_hbm.at[wid], dst_v)
    plsc.subcore_barrier()

    def chunk(j, carry):
        pltpu.async_copy(hn_hbm.at[src_v.at[j]], rows_v, sg0).wait()
        pltpu.sync_copy(rows_v, acc.at[dst_v.at[j]], add=True)
        return carry

    lax.fori_loop(0, NCHUNK, chunk, 0, unroll=False)

    plsc.subcore_barrier()
    pltpu.sync_copy(acc.at[pl.ds(row0, ROWS_PER_TILE)],
                    out_hbm.at[c, pl.ds(row0, ROWS_PER_TILE)])


@functools.cache
def _sc_aggregate():
    return functools.partial(
        pl.kernel,
        out_type=jax.ShapeDtypeStruct((NC, ACC_ROWS, D_C), jnp.float32),
        mesh=plsc.VectorSubcoreMesh(core_axis_name="c", subcore_axis_name="s",
                                    num_cores=NC, num_subcores=NS),
        scratch_types=[
            pltpu.VMEM((NCHUNK, CH), jnp.int32),
            pltpu.VMEM((NCHUNK, CH), jnp.int32),
            pltpu.VMEM((CH, D_C), jnp.float32),
            pltpu.VMEM_SHARED((ACC_ROWS, D_C), jnp.float32),
            pltpu.SemaphoreType.DMA,
        ],
    )(_sc_body)


def _mlp_body(hn_ref, p_ref, w1a_ref, w1b_ref, b1_ref, w2_ref, b2_ref,
              w3_ref, b3_ref, out_ref):
    aggr = p_ref[0] + p_ref[1]
    hi = lax.Precision.DEFAULT
    h = jnp.dot(hn_ref[...], w1a_ref[...], precision=hi,
                preferred_element_type=jnp.float32)
    h += jnp.dot(aggr, w1b_ref[...], precision=hi,
                 preferred_element_type=jnp.float32)
    h = jnp.maximum(h + b1_ref[...], 0.0)
    h = jnp.maximum(
        jnp.dot(h, w2_ref[...], precision=hi,
                preferred_element_type=jnp.float32) + b2_ref[...], 0.0)
    out_ref[...] = jnp.dot(h, w3_ref[...], precision=hi,
                           preferred_element_type=jnp.float32) + b3_ref[...]


def _mlp(hn, partials, w1a, w1b, b1, w2, b2, w3, b3):
    blk = 2000
    grid = (N_NODES_C // blk,)
    wspec = pl.BlockSpec((D_C, D_C), lambda i: (0, 0))
    bspec = pl.BlockSpec((1, D_C), lambda i: (0, 0))
    return pl.pallas_call(
        _mlp_body,
        grid=grid,
        in_specs=[
            pl.BlockSpec((blk, D_C), lambda i: (i, 0)),
            pl.BlockSpec((NC, blk, D_C), lambda i: (0, i, 0)),
            wspec, wspec, bspec, wspec, bspec, wspec, bspec,
        ],
        out_specs=pl.BlockSpec((blk, D_C), lambda i: (i, 0)),
        out_shape=jax.ShapeDtypeStruct((N_NODES_C, D_C), jnp.float32),
    )(hn, partials, w1a, w1b, b1, w2, b2, w3, b3)


def kernel(hn, edge_index, he, W1, b1, W2, b2, W3, b3):
    del he
    src = edge_index[0]
    dst = edge_index[1]
    pad = E_PAD - N_EDGES_C
    src = jnp.concatenate([src, jnp.zeros((pad,), jnp.int32)])
    dst = jnp.concatenate([dst, jnp.full((pad,), N_NODES_C, jnp.int32)])
    src = src.reshape(NW, NCHUNK, CH)
    dst = dst.reshape(NW, NCHUNK, CH)
    zeros = jnp.zeros((ACC_ROWS, D_C), jnp.float32)

    partials = _sc_aggregate()(src, dst, hn, zeros)

    w1a = W1[:, :D_C].T
    w1b = W1[:, D_C:].T
    return _mlp(hn, partials, w1a, w1b, b1.reshape(1, -1),
                W2.T, b2.reshape(1, -1), W3.T, b3.reshape(1, -1))
